# Initial kernel scaffold; baseline (speedup 1.0000x reference)
import jax
import jax.numpy as jnp
from jax.experimental import pallas as pl


def kernel(lm_logits, beam_scores, decoder_inputs):
    raise NotImplementedError("write your pallas kernel here")


# speedup vs baseline: 80.4048x; 80.4048x over previous
import functools

import jax
import jax.numpy as jnp
from jax import lax
from jax.experimental import pallas as pl
from jax.experimental.pallas import tpu as pltpu
from jax.experimental.pallas import tpu_sc as plsc

NUM_BEAMS = 8
VOCAB = 51289
NW = 32
CHUNKS_PER_BEAM = NW // NUM_BEAMS
CHUNK = 12832
VPAD = CHUNK * CHUNKS_PER_BEAM
NVREG = CHUNK // 16
K = 8
NEG = -1e30
BIGI = 2**30


def _sc_body(x_hbm, cand_v_hbm, cand_i_hbm, stats_hbm, xbuf, vvmem, ivmem, svmem):
    wid = lax.axis_index("s") * 2 + lax.axis_index("c")
    pltpu.sync_copy(x_hbm.at[pl.ds(wid * CHUNK, CHUNK)], xbuf)
    iota = lax.iota(jnp.int32, 16)
    negv = jnp.full((16,), NEG, jnp.float32)
    bigv = jnp.full((16,), BIGI, jnp.int32)

    def insert(i, carry):
        v = xbuf[pl.ds(i * 16, 16)]
        iv = iota + i * 16
        out = []
        for j in range(K):
            r, ridx = carry[j], carry[K + j]
            take = v > r
            out.append((jnp.where(take, v, r), jnp.where(take, iv, ridx)))
            v = jnp.where(take, r, v)
            iv = jnp.where(take, ridx, iv)
        return tuple(o[0] for o in out) + tuple(o[1] for o in out)

    init = (negv,) * K + (bigv,) * K
    carry = lax.fori_loop(0, NVREG, insert, init)
    for j in range(K):
        vvmem[pl.ds(j * 16, 16)] = carry[j]
        ivmem[pl.ds(j * 16, 16)] = carry[K + j]
    m = carry[0]

    def sumexp(i, s):
        return s + jnp.exp(xbuf[pl.ds(i * 16, 16)] - m)

    s = lax.fori_loop(0, NVREG, sumexp, jnp.zeros((16,), jnp.float32))
    svmem[pl.ds(0, 16)] = m
    svmem[pl.ds(16, 16)] = s
    pltpu.sync_copy(vvmem, cand_v_hbm.at[wid])
    pltpu.sync_copy(ivmem, cand_i_hbm.at[wid])
    pltpu.sync_copy(svmem, stats_hbm.at[wid])


@functools.lru_cache(maxsize=1)
def _sc_scan():
    return pl.kernel(
        _sc_body,
        out_type=[
            jax.ShapeDtypeStruct((NW, K * 16), jnp.float32),
            jax.ShapeDtypeStruct((NW, K * 16), jnp.int32),
            jax.ShapeDtypeStruct((NW, 32), jnp.float32),
        ],
        mesh=plsc.VectorSubcoreMesh(core_axis_name="c", subcore_axis_name="s"),
        scratch_types=[
            pltpu.VMEM((CHUNK,), jnp.float32),
            pltpu.VMEM((K * 16,), jnp.float32),
            pltpu.VMEM((K * 16,), jnp.int32),
            pltpu.VMEM((32,), jnp.float32),
        ],
    )


def _tc_merge(cv_ref, ci_ref, m_ref, s_ref, bs_ref, dec_ref,
              reord_ref, sc_ref, tok_ref, bidx_ref):
    m_all = m_ref[:, :]
    s_all = s_ref[:, :]
    mb = jnp.max(m_all, axis=1, keepdims=True)
    sb = jnp.sum(s_all * jnp.exp(m_all - mb), axis=1, keepdims=True)
    lse = jnp.log(sb) + mb

    cv = cv_ref[:, :]
    ci = ci_ref[:, :]
    col = lax.broadcasted_iota(jnp.int32, (NUM_BEAMS, 4 * K * 16), 1)
    row = lax.broadcasted_iota(jnp.int32, (NUM_BEAMS, 4 * K * 16), 0)
    tok = ci + (col // (K * 16)) * CHUNK
    flat = row * VOCAB + tok
    adj = jnp.where(tok < VOCAB, cv - lse + bs_ref[:, :], NEG)

    io8 = lax.broadcasted_iota(jnp.int32, (1, K), 1)
    sc = jnp.zeros((1, K), jnp.float32)
    fl = jnp.zeros((1, K), jnp.int32)
    flats = []
    work = adj
    for j in range(K):
        vmax = jnp.max(work)
        fmin = jnp.min(jnp.where(work == vmax, flat, BIGI))
        work = jnp.where(flat == fmin, NEG, work)
        sc = jnp.where(io8 == j, vmax, sc)
        fl = jnp.where(io8 == j, fmin, fl)
        flats.append(fmin)

    sc_ref[:, :] = sc
    tok_ref[:, :] = fl % VOCAB
    bidx_ref[:, :] = fl // VOCAB

    orow = lax.broadcasted_iota(jnp.int32, (NUM_BEAMS, 1), 0)
    bi_rows = jnp.zeros((NUM_BEAMS, 1), jnp.int32)
    for j in range(K):
        bi_rows = jnp.where(orow == j, flats[j] // VOCAB, bi_rows)
    reord = jnp.zeros(dec_ref.shape, jnp.int32)
    for k in range(NUM_BEAMS):
        reord = jnp.where(bi_rows == k, dec_ref[k:k + 1, :], reord)
    reord_ref[:, :] = reord


def kernel(lm_logits, beam_scores, decoder_inputs):
    x = lm_logits[:, -1, :]
    xpad = jnp.pad(x, ((0, 0), (0, VPAD - VOCAB)), constant_values=NEG)
    cand_v, cand_i, stats = _sc_scan()(jnp.reshape(xpad, (-1,)))

    m_all = jnp.reshape(stats[:, :16], (NUM_BEAMS, 64))
    s_all = jnp.reshape(stats[:, 16:], (NUM_BEAMS, 64))
    cv = jnp.reshape(cand_v, (NUM_BEAMS, 4 * K * 16))
    ci = jnp.reshape(cand_i, (NUM_BEAMS, 4 * K * 16))
    bs = jnp.reshape(beam_scores, (NUM_BEAMS, 1))

    seq = decoder_inputs.shape[1]
    reord, sc, tok, bidx = pl.pallas_call(
        _tc_merge,
        out_shape=[
            jax.ShapeDtypeStruct((NUM_BEAMS, seq), jnp.int32),
            jax.ShapeDtypeStruct((1, K), jnp.float32),
            jax.ShapeDtypeStruct((1, K), jnp.int32),
            jax.ShapeDtypeStruct((1, K), jnp.int32),
        ],
    )(cv, ci, m_all, s_all, bs, decoder_inputs)

    new_decoder_inputs = jnp.concatenate(
        [reord, jnp.reshape(tok, (NUM_BEAMS, 1))], axis=1)
    return (new_decoder_inputs, jnp.reshape(sc, (NUM_BEAMS,)),
            jnp.reshape(tok, (NUM_BEAMS,)), jnp.reshape(bidx, (NUM_BEAMS,)))
